# baseline (speedup 1.0000x reference)
import functools

import jax
import jax.numpy as jnp
from jax import lax
from jax.experimental import pallas as pl
from jax.experimental.pallas import tpu as pltpu
from jax.experimental.pallas import tpu_sc as plsc

TOP_K = 100
LANE = 128
SUB = 16
ROWS = 8
KEEP_C = 128
KEEP_S = 128
NCORES = 2
NSUBCORES = 16
NW = NCORES * NSUBCORES


def _stage_a_kernel(x_ref, sm_ref, cid_ref, gflat_ref, *, nchunks, block_rows):
    BIG = jnp.int32(2**30)
    i = pl.program_id(0)
    x = x_ref[...]
    x3 = x.reshape(block_rows, nchunks, LANE)
    sms = [jnp.max(x3[:, :, j * SUB:(j + 1) * SUB], axis=2) for j in range(8)]
    sm = jnp.stack(sms, axis=2)
    cm = jnp.max(sm, axis=2)
    pad = jnp.full((block_rows, nchunks, 8), -jnp.inf, jnp.float32)
    sm_ref[...] = jnp.concatenate([sm, pad], axis=2).reshape(block_rows, nchunks * SUB)

    pos = lax.broadcasted_iota(jnp.int32, (block_rows, nchunks), 1)
    liota = lax.broadcasted_iota(jnp.int32, (block_rows, LANE), 1)

    def step(j, carry):
        cm, acc = carry
        m = jnp.max(cm, axis=1, keepdims=True)
        key = jnp.where(cm == m, pos, BIG)
        p = jnp.min(key, axis=1, keepdims=True)
        acc = jnp.where(liota == j, p, acc)
        cm = jnp.where(key == p, -jnp.inf, cm)
        return cm, acc

    _, cids = lax.fori_loop(0, KEEP_C, step, (cm, jnp.zeros((block_rows, LANE), jnp.int32)))
    cid_ref[...] = cids
    row = lax.broadcasted_iota(jnp.int32, (block_rows, LANE), 0) + i * block_rows
    gflat_ref[...] = cids + row * nchunks


def _stage_c_kernel(smg_ref, cid_ref, ids_ref, gy_ref, *, nchunks, nsub, block_rows):
    BIG = jnp.int32(2**30)
    i = pl.program_id(0)
    sm = smg_ref[...][:, :, :8].reshape(block_rows, KEEP_C * 8)
    cids = cid_ref[...]
    cid8 = jnp.broadcast_to(cids[:, :, None], (block_rows, KEEP_C, 8))
    sub_i = lax.broadcasted_iota(jnp.int32, (block_rows, KEEP_C, 8), 2)
    fullmap = (cid8 * 8 + sub_i).reshape(block_rows, KEEP_C * 8)

    pos = lax.broadcasted_iota(jnp.int32, (block_rows, KEEP_C * 8), 1)
    liota = lax.broadcasted_iota(jnp.int32, (block_rows, LANE), 1)

    def step(j, carry):
        sm, acc = carry
        m = jnp.max(sm, axis=1, keepdims=True)
        key = jnp.where(sm == m, pos, BIG)
        p = jnp.min(key, axis=1, keepdims=True)
        sel = key == p
        fs = jnp.min(jnp.where(sel, fullmap, BIG), axis=1, keepdims=True)
        acc = jnp.where(liota == j, fs, acc)
        sm = jnp.where(sel, -jnp.inf, sm)
        return sm, acc

    _, ids = lax.fori_loop(0, KEEP_S, step, (sm, jnp.zeros((block_rows, LANE), jnp.int32)))
    ids_ref[...] = ids
    row = lax.broadcasted_iota(jnp.int32, (block_rows, LANE), 0) + i * block_rows
    gy_ref[...] = ids + row * nsub


def _stage_e_kernel(candy_ref, candw_ref, ids_ref, words_ref, scores_ref, *, k, block_rows):
    BIG = jnp.int32(2**30)
    x = candy_ref[...]
    cw = candw_ref[...]
    ids = ids_ref[...]
    n = KEEP_S * SUB
    l16 = lax.broadcasted_iota(jnp.int32, (block_rows, KEEP_S, SUB), 2)
    origmap = (jnp.broadcast_to(ids[:, :, None], (block_rows, KEEP_S, SUB)) * SUB
               + l16).reshape(block_rows, n)
    liota = lax.broadcasted_iota(jnp.int32, (block_rows, LANE), 1)

    def step(j, carry):
        x, wacc, sacc = carry
        m = jnp.max(x, axis=1, keepdims=True)
        key = jnp.where(x == m, origmap, BIG)
        om = jnp.min(key, axis=1, keepdims=True)
        sel = key == om
        w = jnp.min(jnp.where(sel, cw, BIG), axis=1, keepdims=True)
        wacc = jnp.where(liota == j, w, wacc)
        sacc = jnp.where(liota == j, m, sacc)
        x = jnp.where(sel, -jnp.inf, x)
        return x, wacc, sacc

    init = (x, jnp.zeros((block_rows, LANE), jnp.int32),
            jnp.zeros((block_rows, LANE), jnp.float32))
    _, wacc, sacc = lax.fori_loop(0, k, step, init)
    words_ref[...] = wacc
    scores_ref[...] = sacc


def _sc_gather_rows(table, idx2):
    nj, _ = idx2.shape
    n = nj * LANE
    d = table.shape[1]
    bpw = n // NW
    jw = bpw // LANE
    mesh = plsc.VectorSubcoreMesh(core_axis_name="c", subcore_axis_name="s")

    @functools.partial(
        pl.kernel, mesh=mesh,
        out_type=jax.ShapeDtypeStruct((n, d), table.dtype),
        compiler_params=pltpu.CompilerParams(use_tc_tiling_on_sc=False),
        scratch_types=[
            pltpu.VMEM((jw, LANE), jnp.int32),
            pltpu.VMEM((bpw, d), table.dtype),
            pltpu.SemaphoreType.DMA,
        ],
    )
    def k(table_hbm, idx_hbm, out_hbm, idx_v, rows_v, sem):
        wid = lax.axis_index("s") * NCORES + lax.axis_index("c")
        pltpu.sync_copy(idx_hbm.at[pl.ds(wid * jw, jw)], idx_v)
        copies = [
            pltpu.make_async_copy(
                table_hbm.at[idx_v.at[j]], rows_v.at[pl.ds(j * LANE, LANE)], sem)
            for j in range(jw)
        ]
        for c in copies:
            c.start()
        for c in copies:
            c.wait()
        pltpu.sync_copy(rows_v, out_hbm.at[pl.ds(wid * bpw, bpw)])

    return k(table, idx2)


def kernel(y_pred, word_table):
    batch, vocab = y_pred.shape
    nchunks = (vocab + LANE - 1) // LANE
    vp = nchunks * LANE
    nsub = vp // SUB
    x = jnp.pad(y_pred, ((0, 0), (0, vp - vocab)), constant_values=-jnp.inf)
    wt = jnp.pad(word_table, (0, vp - vocab))

    nblocks = batch // ROWS
    submax, cids, gflat = pl.pallas_call(
        functools.partial(_stage_a_kernel, nchunks=nchunks, block_rows=ROWS),
        grid=(nblocks,),
        in_specs=[pl.BlockSpec((ROWS, vp), lambda i: (i, 0))],
        out_specs=[
            pl.BlockSpec((ROWS, nchunks * SUB), lambda i: (i, 0)),
            pl.BlockSpec((ROWS, LANE), lambda i: (i, 0)),
            pl.BlockSpec((ROWS, LANE), lambda i: (i, 0)),
        ],
        out_shape=[
            jax.ShapeDtypeStruct((batch, nchunks * SUB), jnp.float32),
            jax.ShapeDtypeStruct((batch, LANE), jnp.int32),
            jax.ShapeDtypeStruct((batch, LANE), jnp.int32),
        ],
    )(x)

    sub_table = submax.reshape(batch * nchunks, SUB)
    smg = _sc_gather_rows(sub_table, gflat).reshape(batch, KEEP_C, SUB)

    ids_sub, gy = pl.pallas_call(
        functools.partial(_stage_c_kernel, nchunks=nchunks, nsub=nsub, block_rows=ROWS),
        grid=(nblocks,),
        in_specs=[
            pl.BlockSpec((ROWS, KEEP_C, SUB), lambda i: (i, 0, 0)),
            pl.BlockSpec((ROWS, LANE), lambda i: (i, 0)),
        ],
        out_specs=[
            pl.BlockSpec((ROWS, LANE), lambda i: (i, 0)),
            pl.BlockSpec((ROWS, LANE), lambda i: (i, 0)),
        ],
        out_shape=[
            jax.ShapeDtypeStruct((batch, LANE), jnp.int32),
            jax.ShapeDtypeStruct((batch, LANE), jnp.int32),
        ],
    )(smg, cids)

    y_sub = x.reshape(batch * nsub, SUB)
    wt_sub = wt.reshape(nsub, SUB)
    candy = _sc_gather_rows(y_sub, gy).reshape(batch, KEEP_S * SUB)
    candw = _sc_gather_rows(wt_sub, ids_sub).reshape(batch, KEEP_S * SUB)

    words, scores = pl.pallas_call(
        functools.partial(_stage_e_kernel, k=TOP_K, block_rows=ROWS),
        grid=(nblocks,),
        in_specs=[
            pl.BlockSpec((ROWS, KEEP_S * SUB), lambda i: (i, 0)),
            pl.BlockSpec((ROWS, KEEP_S * SUB), lambda i: (i, 0)),
            pl.BlockSpec((ROWS, LANE), lambda i: (i, 0)),
        ],
        out_specs=[
            pl.BlockSpec((ROWS, LANE), lambda i: (i, 0)),
            pl.BlockSpec((ROWS, LANE), lambda i: (i, 0)),
        ],
        out_shape=[
            jax.ShapeDtypeStruct((batch, LANE), jnp.int32),
            jax.ShapeDtypeStruct((batch, LANE), jnp.float32),
        ],
    )(candy, candw, ids_sub)

    return words[:, :TOP_K], scores[:, :TOP_K]


# speedup vs baseline: 10.6506x; 10.6506x over previous
import functools

import jax
import jax.numpy as jnp
from jax import lax
from jax.experimental import pallas as pl
from jax.experimental.pallas import tpu as pltpu
from jax.experimental.pallas import tpu_sc as plsc

TOP_K = 100
LANE = 128
SUB = 16
ROWS = 8
KEEP_C = 128
KEEP_S = 128
NCORES = 2
NSUBCORES = 16
NW = NCORES * NSUBCORES


def _sweep_kernel(x_ref, sm_ref, cm_ref, *, nchunks, block_rows):
    x = x_ref[...]
    sm = jnp.max(x.reshape(block_rows, nchunks * 8, SUB), axis=2)
    cm = jnp.max(sm.reshape(block_rows, nchunks, 8), axis=2)
    sm3 = sm.reshape(block_rows, nchunks, 8)
    pad = jnp.full((block_rows, nchunks, 8), -jnp.inf, jnp.float32)
    sm_ref[...] = jnp.concatenate([sm3, pad], axis=2).reshape(block_rows * nchunks, SUB)
    cm_ref[...] = cm


def _chunk_select_kernel(cm_ref, cid_ref, gflat_ref, *, nchunks, batch):
    BIG = jnp.int32(2**30)
    cm = cm_ref[...]
    pos = lax.broadcasted_iota(jnp.int32, (batch, nchunks), 1)
    liota = lax.broadcasted_iota(jnp.int32, (batch, LANE), 1)

    def step(j, carry):
        cm, acc = carry
        m = jnp.max(cm, axis=1, keepdims=True)
        key = jnp.where(cm == m, pos, BIG)
        p = jnp.min(key, axis=1, keepdims=True)
        acc = jnp.where(liota == j, p, acc)
        cm = jnp.where(key == p, -jnp.inf, cm)
        return cm, acc

    _, cids = lax.fori_loop(0, KEEP_C, step, (cm, jnp.zeros((batch, LANE), jnp.int32)))
    cid_ref[...] = cids
    row = lax.broadcasted_iota(jnp.int32, (batch, LANE), 0)
    gflat_ref[...] = cids + row * nchunks


def _sub_select_kernel(smg_ref, cid_ref, ids_ref, gy_ref, *, nsub, batch):
    BIG = jnp.int32(2**30)
    sm = smg_ref[...][:, :, :8].reshape(batch, KEEP_C * 8)
    cids = cid_ref[...]
    cid8 = jnp.broadcast_to(cids[:, :, None], (batch, KEEP_C, 8))
    sub_i = lax.broadcasted_iota(jnp.int32, (batch, KEEP_C, 8), 2)
    fullmap = (cid8 * 8 + sub_i).reshape(batch, KEEP_C * 8)

    pos = lax.broadcasted_iota(jnp.int32, (batch, KEEP_C * 8), 1)
    liota = lax.broadcasted_iota(jnp.int32, (batch, LANE), 1)

    def step(j, carry):
        sm, acc = carry
        m = jnp.max(sm, axis=1, keepdims=True)
        key = jnp.where(sm == m, pos, BIG)
        p = jnp.min(key, axis=1, keepdims=True)
        sel = key == p
        fs = jnp.min(jnp.where(sel, fullmap, BIG), axis=1, keepdims=True)
        acc = jnp.where(liota == j, fs, acc)
        sm = jnp.where(sel, -jnp.inf, sm)
        return sm, acc

    _, ids = lax.fori_loop(0, KEEP_S, step, (sm, jnp.zeros((batch, LANE), jnp.int32)))
    ids_ref[...] = ids
    row = lax.broadcasted_iota(jnp.int32, (batch, LANE), 0)
    gy_ref[...] = ids + row * nsub


def _final_kernel(candy_ref, candw_ref, ids_ref, words_ref, scores_ref, *, k, batch):
    BIG = jnp.int32(2**30)
    x = candy_ref[...]
    cw = candw_ref[...]
    ids = ids_ref[...]
    n = KEEP_S * SUB
    l16 = lax.broadcasted_iota(jnp.int32, (batch, KEEP_S, SUB), 2)
    origmap = (jnp.broadcast_to(ids[:, :, None], (batch, KEEP_S, SUB)) * SUB
               + l16).reshape(batch, n)
    liota = lax.broadcasted_iota(jnp.int32, (batch, LANE), 1)

    def step(j, carry):
        x, wacc, sacc = carry
        m = jnp.max(x, axis=1, keepdims=True)
        key = jnp.where(x == m, origmap, BIG)
        om = jnp.min(key, axis=1, keepdims=True)
        sel = key == om
        w = jnp.min(jnp.where(sel, cw, BIG), axis=1, keepdims=True)
        wacc = jnp.where(liota == j, w, wacc)
        sacc = jnp.where(liota == j, m, sacc)
        x = jnp.where(sel, -jnp.inf, x)
        return x, wacc, sacc

    init = (x, jnp.zeros((batch, LANE), jnp.int32),
            jnp.zeros((batch, LANE), jnp.float32))
    _, wacc, sacc = lax.fori_loop(0, k, step, init)
    words_ref[...] = wacc
    scores_ref[...] = sacc


def _sc_gather_rows(table, idx2):
    nj, _ = idx2.shape
    n = nj * LANE
    d = table.shape[1]
    bpw = n // NW
    jw = bpw // LANE
    mesh = plsc.VectorSubcoreMesh(core_axis_name="c", subcore_axis_name="s")

    @functools.partial(
        pl.kernel, mesh=mesh,
        out_type=jax.ShapeDtypeStruct((n, d), table.dtype),
        compiler_params=pltpu.CompilerParams(use_tc_tiling_on_sc=False),
        scratch_types=[
            pltpu.VMEM((jw, LANE), jnp.int32),
            pltpu.VMEM((bpw, d), table.dtype),
            pltpu.SemaphoreType.DMA,
        ],
    )
    def k(table_hbm, idx_hbm, out_hbm, idx_v, rows_v, sem):
        wid = lax.axis_index("s") * NCORES + lax.axis_index("c")
        pltpu.sync_copy(idx_hbm.at[pl.ds(wid * jw, jw)], idx_v)
        copies = [
            pltpu.make_async_copy(
                table_hbm.at[idx_v.at[j]], rows_v.at[pl.ds(j * LANE, LANE)], sem)
            for j in range(jw)
        ]
        for c in copies:
            c.start()
        for c in copies:
            c.wait()
        pltpu.sync_copy(rows_v, out_hbm.at[pl.ds(wid * bpw, bpw)])

    return k(table, idx2)


def kernel(y_pred, word_table):
    batch, vocab = y_pred.shape
    nchunks = (vocab + LANE - 1) // LANE
    vp = nchunks * LANE
    nsub = vp // SUB
    x = jnp.pad(y_pred, ((0, 0), (0, vp - vocab)), constant_values=-jnp.inf)
    wt = jnp.pad(word_table, (0, vp - vocab))

    nblocks = batch // ROWS
    sub_table, cmax = pl.pallas_call(
        functools.partial(_sweep_kernel, nchunks=nchunks, block_rows=ROWS),
        grid=(nblocks,),
        in_specs=[pl.BlockSpec((ROWS, vp), lambda i: (i, 0))],
        out_specs=[
            pl.BlockSpec((ROWS * nchunks, SUB), lambda i: (i, 0)),
            pl.BlockSpec((ROWS, nchunks), lambda i: (i, 0)),
        ],
        out_shape=[
            jax.ShapeDtypeStruct((batch * nchunks, SUB), jnp.float32),
            jax.ShapeDtypeStruct((batch, nchunks), jnp.float32),
        ],
    )(x)

    cids, gflat = pl.pallas_call(
        functools.partial(_chunk_select_kernel, nchunks=nchunks, batch=batch),
        out_shape=[
            jax.ShapeDtypeStruct((batch, LANE), jnp.int32),
            jax.ShapeDtypeStruct((batch, LANE), jnp.int32),
        ],
    )(cmax)

    smg = _sc_gather_rows(sub_table, gflat).reshape(batch, KEEP_C, SUB)

    ids_sub, gy = pl.pallas_call(
        functools.partial(_sub_select_kernel, nsub=nsub, batch=batch),
        out_shape=[
            jax.ShapeDtypeStruct((batch, LANE), jnp.int32),
            jax.ShapeDtypeStruct((batch, LANE), jnp.int32),
        ],
    )(smg, cids)

    y_sub = x.reshape(batch * nsub, SUB)
    wt_sub = wt.reshape(nsub, SUB)
    candy = _sc_gather_rows(y_sub, gy).reshape(batch, KEEP_S * SUB)
    candw = _sc_gather_rows(wt_sub, ids_sub).reshape(batch, KEEP_S * SUB)

    words, scores = pl.pallas_call(
        functools.partial(_final_kernel, k=TOP_K, batch=batch),
        out_shape=[
            jax.ShapeDtypeStruct((batch, LANE), jnp.int32),
            jax.ShapeDtypeStruct((batch, LANE), jnp.float32),
        ],
    )(candy, candw, ids_sub)

    return words[:, :TOP_K], scores[:, :TOP_K]


# speedup vs baseline: 18.1632x; 1.7054x over previous
import functools

import jax
import jax.numpy as jnp
from jax import lax
from jax.experimental import pallas as pl
from jax.experimental.pallas import tpu as pltpu
from jax.experimental.pallas import tpu_sc as plsc

TOP_K = 100
LANE = 128
SUB = 16
ROWS = 8
KEEP_C = 128
KEEP_S = 128
NCORES = 2
NSUBCORES = 16
NW = NCORES * NSUBCORES


def _sweep_kernel(x_ref, cm_ref, *, nchunks, block_rows):
    x = x_ref[...]
    cm_ref[...] = jnp.max(x.reshape(block_rows, nchunks, LANE), axis=2)


def _chunk_select_kernel(cm_ref, cid_ref, gflat_ref, *, nchunks, batch):
    BIG = jnp.int32(2**30)
    cm = cm_ref[...]
    pos = lax.broadcasted_iota(jnp.int32, (batch, nchunks), 1)
    liota = lax.broadcasted_iota(jnp.int32, (batch, LANE), 1)

    def step(j, carry):
        cm, acc = carry
        m = jnp.max(cm, axis=1, keepdims=True)
        key = jnp.where(cm == m, pos, BIG)
        p = jnp.min(key, axis=1, keepdims=True)
        acc = jnp.where(liota == j, p, acc)
        cm = jnp.where(key == p, -jnp.inf, cm)
        return cm, acc

    _, cids = lax.fori_loop(0, KEEP_C, step, (cm, jnp.zeros((batch, LANE), jnp.int32)))
    cid_ref[...] = cids
    row = lax.broadcasted_iota(jnp.int32, (batch, LANE), 0)
    gflat_ref[...] = cids + row * nchunks


def _submax_kernel(cube_ref, sm_ref, *, block_rows):
    cube = cube_ref[...]
    sm_ref[...] = jnp.max(
        cube.reshape(block_rows, KEEP_C, 8, SUB), axis=3).reshape(block_rows, KEEP_C * 8)


def _sub_select_kernel(sm_ref, cid_ref, ids_ref, gy_ref, *, nsub, batch):
    BIG = jnp.int32(2**30)
    sm = sm_ref[...]
    cids = cid_ref[...]
    cid8 = jnp.broadcast_to(cids[:, :, None], (batch, KEEP_C, 8))
    sub_i = lax.broadcasted_iota(jnp.int32, (batch, KEEP_C, 8), 2)
    fullmap = (cid8 * 8 + sub_i).reshape(batch, KEEP_C * 8)

    pos = lax.broadcasted_iota(jnp.int32, (batch, KEEP_C * 8), 1)
    liota = lax.broadcasted_iota(jnp.int32, (batch, LANE), 1)

    def step(j, carry):
        sm, acc = carry
        m = jnp.max(sm, axis=1, keepdims=True)
        key = jnp.where(sm == m, pos, BIG)
        p = jnp.min(key, axis=1, keepdims=True)
        sel = key == p
        fs = jnp.min(jnp.where(sel, fullmap, BIG), axis=1, keepdims=True)
        acc = jnp.where(liota == j, fs, acc)
        sm = jnp.where(sel, -jnp.inf, sm)
        return sm, acc

    _, ids = lax.fori_loop(0, KEEP_S, step, (sm, jnp.zeros((batch, LANE), jnp.int32)))
    ids_ref[...] = ids
    row = lax.broadcasted_iota(jnp.int32, (batch, LANE), 0)
    gy_ref[...] = ids + row * nsub


def _final_kernel(candy_ref, candw_ref, ids_ref, words_ref, scores_ref, *, k, batch):
    BIG = jnp.int32(2**30)
    x = candy_ref[...]
    cw = candw_ref[...]
    ids = ids_ref[...]
    n = KEEP_S * SUB
    l16 = lax.broadcasted_iota(jnp.int32, (batch, KEEP_S, SUB), 2)
    origmap = (jnp.broadcast_to(ids[:, :, None], (batch, KEEP_S, SUB)) * SUB
               + l16).reshape(batch, n)
    liota = lax.broadcasted_iota(jnp.int32, (batch, LANE), 1)

    def step(j, carry):
        x, wacc, sacc = carry
        m = jnp.max(x, axis=1, keepdims=True)
        key = jnp.where(x == m, origmap, BIG)
        om = jnp.min(key, axis=1, keepdims=True)
        sel = key == om
        w = jnp.min(jnp.where(sel, cw, BIG), axis=1, keepdims=True)
        wacc = jnp.where(liota == j, w, wacc)
        sacc = jnp.where(liota == j, m, sacc)
        x = jnp.where(sel, -jnp.inf, x)
        return x, wacc, sacc

    init = (x, jnp.zeros((batch, LANE), jnp.int32),
            jnp.zeros((batch, LANE), jnp.float32))
    _, wacc, sacc = lax.fori_loop(0, k, step, init)
    words_ref[...] = wacc
    scores_ref[...] = sacc


def _sc_gather_rows(table, idx2):
    nj, _ = idx2.shape
    n = nj * LANE
    d = table.shape[1]
    bpw = n // NW
    jw = bpw // LANE
    mesh = plsc.VectorSubcoreMesh(core_axis_name="c", subcore_axis_name="s")

    @functools.partial(
        pl.kernel, mesh=mesh,
        out_type=jax.ShapeDtypeStruct((n, d), table.dtype),
        compiler_params=pltpu.CompilerParams(use_tc_tiling_on_sc=False),
        scratch_types=[
            pltpu.VMEM((jw, LANE), jnp.int32),
            pltpu.VMEM((bpw, d), table.dtype),
            pltpu.SemaphoreType.DMA,
        ],
    )
    def k(table_hbm, idx_hbm, out_hbm, idx_v, rows_v, sem):
        wid = lax.axis_index("s") * NCORES + lax.axis_index("c")
        pltpu.sync_copy(idx_hbm.at[pl.ds(wid * jw, jw)], idx_v)
        copies = [
            pltpu.make_async_copy(
                table_hbm.at[idx_v.at[j]], rows_v.at[pl.ds(j * LANE, LANE)], sem)
            for j in range(jw)
        ]
        for c in copies:
            c.start()
        for c in copies:
            c.wait()
        pltpu.sync_copy(rows_v, out_hbm.at[pl.ds(wid * bpw, bpw)])

    return k(table, idx2)


def kernel(y_pred, word_table):
    batch, vocab = y_pred.shape
    nchunks = (vocab + LANE - 1) // LANE
    vp = nchunks * LANE
    nsub = vp // SUB
    x = jnp.pad(y_pred, ((0, 0), (0, vp - vocab)), constant_values=-jnp.inf)
    wt = jnp.pad(word_table, (0, vp - vocab))

    nblocks = batch // ROWS
    cmax = pl.pallas_call(
        functools.partial(_sweep_kernel, nchunks=nchunks, block_rows=ROWS),
        grid=(nblocks,),
        in_specs=[pl.BlockSpec((ROWS, vp), lambda i: (i, 0))],
        out_specs=pl.BlockSpec((ROWS, nchunks), lambda i: (i, 0)),
        out_shape=jax.ShapeDtypeStruct((batch, nchunks), jnp.float32),
    )(x)

    cids, gflat = pl.pallas_call(
        functools.partial(_chunk_select_kernel, nchunks=nchunks, batch=batch),
        out_shape=[
            jax.ShapeDtypeStruct((batch, LANE), jnp.int32),
            jax.ShapeDtypeStruct((batch, LANE), jnp.int32),
        ],
    )(cmax)

    chunk_table = x.reshape(batch * nchunks, LANE)
    cube = _sc_gather_rows(chunk_table, gflat).reshape(batch, KEEP_C, LANE)

    sm = pl.pallas_call(
        functools.partial(_submax_kernel, block_rows=ROWS),
        grid=(nblocks,),
        in_specs=[pl.BlockSpec((ROWS, KEEP_C, LANE), lambda i: (i, 0, 0))],
        out_specs=pl.BlockSpec((ROWS, KEEP_C * 8), lambda i: (i, 0)),
        out_shape=jax.ShapeDtypeStruct((batch, KEEP_C * 8), jnp.float32),
    )(cube)

    ids_sub, gy = pl.pallas_call(
        functools.partial(_sub_select_kernel, nsub=nsub, batch=batch),
        out_shape=[
            jax.ShapeDtypeStruct((batch, LANE), jnp.int32),
            jax.ShapeDtypeStruct((batch, LANE), jnp.int32),
        ],
    )(sm, cids)

    y_sub = x.reshape(batch * nsub, SUB)
    wt_sub = wt.reshape(nsub, SUB)
    candy = _sc_gather_rows(y_sub, gy).reshape(batch, KEEP_S * SUB)
    candw = _sc_gather_rows(wt_sub, ids_sub).reshape(batch, KEEP_S * SUB)

    words, scores = pl.pallas_call(
        functools.partial(_final_kernel, k=TOP_K, batch=batch),
        out_shape=[
            jax.ShapeDtypeStruct((batch, LANE), jnp.int32),
            jax.ShapeDtypeStruct((batch, LANE), jnp.float32),
        ],
    )(candy, candw, ids_sub)

    return words[:, :TOP_K], scores[:, :TOP_K]
